# baseline (speedup 1.0000x reference)
import functools

import jax
import jax.numpy as jnp
from jax import lax
from jax.experimental import pallas as pl
from jax.experimental.pallas import tpu as pltpu
from jax.experimental.pallas import tpu_sc as plsc

_NC = 2
_NS = 16
_NW = _NC * _NS
_CHUNK = 128
_NBUF = 10
_LOOKAHEAD = _NBUF // 2
_NTB = 2


@functools.lru_cache(maxsize=None)
def _make_gather(batch, hist, embed):
    n_chunks = hist
    n_groups = n_chunks // _NBUF
    eb = embed // 8
    mesh = plsc.VectorSubcoreMesh(core_axis_name="c", subcore_axis_name="s")

    @functools.partial(
        pl.kernel,
        mesh=mesh,
        out_type=jax.ShapeDtypeStruct((hist, eb, _NW, 8, _CHUNK), jnp.float32),
        scratch_types=[
            pltpu.VMEM((n_chunks, _CHUNK), jnp.int32),
            [pltpu.VMEM((_CHUNK, embed), jnp.float32) for _ in range(_NBUF)],
            [pltpu.VMEM((eb, 8, _CHUNK), jnp.float32) for _ in range(_NTB)],
            [pltpu.SemaphoreType.DMA for _ in range(_NBUF)],
            [pltpu.SemaphoreType.DMA for _ in range(_NTB)],
        ],
        compiler_params=pltpu.CompilerParams(
            use_tc_tiling_on_sc=False, needs_layout_passes=False),
    )
    def gather(idx_hbm, table_hbm, out_hbm, idx_v, gbufs, tbufs, gsems, osems):
        wid = lax.axis_index("s") * _NC + lax.axis_index("c")
        pltpu.sync_copy(idx_hbm.at[wid], idx_v)

        lane = lax.iota(jnp.int32, 16)

        def start_gather(j, b):
            pltpu.async_copy(table_hbm.at[idx_v.at[j]], gbufs[b], gsems[b])

        def wait_gather(j, b):
            pltpu.make_async_copy(
                table_hbm.at[idx_v.at[j]], gbufs[b], gsems[b]).wait()

        def out_slice(j):
            return out_hbm.at[j, :, wid]

        def start_out(j, tb):
            pltpu.async_copy(tbufs[tb], out_slice(j), osems[tb])

        def wait_out(j, tb):
            pltpu.make_async_copy(tbufs[tb], out_slice(j), osems[tb]).wait()

        def transpose(b, tb):
            gb, to = gbufs[b], tbufs[tb]

            def body(ebi, carry):
                for r in range(8):
                    e = lane * 0 + (ebi * 8 + r)
                    for c16 in range(8):
                        rows = lane + (c16 * 16)
                        vals = plsc.load_gather(gb, [rows, e])
                        to[ebi, r, pl.ds(c16 * 16, 16)] = vals
                return carry

            lax.fori_loop(0, eb, body, 0)

        for b in range(_LOOKAHEAD):
            start_gather(b, b)

        def group(g, carry):
            jg = g * _NBUF
            for b in range(_NBUF):
                j = jg + b
                tb = b % _NTB
                wait_gather(j, b)
                if b >= _NTB:
                    wait_out(jg + b - _NTB, tb)
                else:
                    @pl.when(g > 0)
                    def _():
                        wait_out(jg + b - _NTB, tb)
                transpose(b, tb)
                start_out(j, tb)
                bn = (b + _LOOKAHEAD) % _NBUF
                if b < _LOOKAHEAD:
                    start_gather(j + _LOOKAHEAD, bn)
                else:
                    @pl.when(g < n_groups - 1)
                    def _():
                        start_gather(j + _LOOKAHEAD, bn)
            return carry

        lax.fori_loop(0, n_groups, group, 0)

        jg = (n_groups - 1) * _NBUF
        for b in range(_NBUF - _NTB, _NBUF):
            wait_out(jg + b, b % _NTB)

    return gather


def kernel(x, word_embed):
    batch, hist = x.shape
    vocab, embed = word_embed.shape
    idx = x.astype(jnp.int32).reshape(_NW, _CHUNK, hist).transpose(0, 2, 1)
    out5 = _make_gather(batch, hist, embed)(idx, word_embed)
    return out5.transpose(2, 4, 0, 1, 3).reshape(batch, hist, embed)


# speedup vs baseline: 1.4255x; 1.4255x over previous
import functools

import jax
import jax.numpy as jnp
from jax import lax
from jax.experimental import pallas as pl
from jax.experimental.pallas import tpu as pltpu
from jax.experimental.pallas import tpu_sc as plsc

_NC = 2
_NS = 16
_NW = _NC * _NS
_CHUNK = 128
_NBUF = 10
_LOOKAHEAD = _NBUF // 2
_NTB = 2


@functools.lru_cache(maxsize=None)
def _make_gather(batch, hist, embed):
    n_chunks = hist
    n_groups = n_chunks // _NBUF
    eb = embed // 8
    mesh = plsc.VectorSubcoreMesh(core_axis_name="c", subcore_axis_name="s")

    @functools.partial(
        pl.kernel,
        mesh=mesh,
        out_type=jax.ShapeDtypeStruct((hist, eb, _NW, 8, _CHUNK), jnp.float32),
        scratch_types=[
            pltpu.VMEM((n_chunks, _CHUNK), jnp.int32),
            [pltpu.VMEM((_CHUNK, embed), jnp.float32) for _ in range(_NBUF)],
            [pltpu.VMEM((eb, 8, _CHUNK), jnp.float32) for _ in range(_NTB)],
            [pltpu.SemaphoreType.DMA for _ in range(_NBUF)],
            [pltpu.SemaphoreType.DMA for _ in range(_NTB)],
        ],
        compiler_params=pltpu.CompilerParams(
            use_tc_tiling_on_sc=False, needs_layout_passes=False),
    )
    def gather(idx_hbm, table_hbm, out_hbm, idx_v, gbufs, tbufs, gsems, osems):
        wid = lax.axis_index("s") * _NC + lax.axis_index("c")
        pltpu.sync_copy(idx_hbm.at[wid], idx_v)

        lane = lax.iota(jnp.int32, 16)

        def start_gather(j, b):
            pltpu.async_copy(table_hbm.at[idx_v.at[j]], gbufs[b], gsems[b])

        def wait_gather(j, b):
            pltpu.make_async_copy(
                table_hbm.at[idx_v.at[j]], gbufs[b], gsems[b]).wait()

        def out_slice(j):
            return out_hbm.at[j, :, wid]

        def start_out(j, tb):
            pltpu.async_copy(tbufs[tb], out_slice(j), osems[tb])

        def wait_out(j, tb):
            pltpu.make_async_copy(tbufs[tb], out_slice(j), osems[tb]).wait()

        rot = [lax.rem(lane + k, 16) for k in range(16)]

        def transpose(b, tb):
            gb, to = gbufs[b], tbufs[tb]

            @plsc.parallel_loop(0, 32, unroll=2)
            def _(u):
                c16 = lax.shift_right_logical(u, 2)
                e0 = lax.shift_left(lax.bitwise_and(u, 3), 4)
                rows = lane + lax.shift_left(c16, 4)
                for k in range(16):
                    cols = rot[k] + e0
                    vals = plsc.load_gather(gb, [rows, cols])
                    ebi = lax.shift_right_logical(cols, 3)
                    rr = lax.bitwise_and(cols, 7)
                    plsc.store_scatter(to, [ebi, rr, rows], vals)

        for b in range(_LOOKAHEAD):
            start_gather(b, b)

        def group(g, carry):
            jg = g * _NBUF
            for b in range(_NBUF):
                j = jg + b
                tb = b % _NTB
                wait_gather(j, b)
                if b >= _NTB:
                    wait_out(jg + b - _NTB, tb)
                else:
                    @pl.when(g > 0)
                    def _():
                        wait_out(jg + b - _NTB, tb)
                transpose(b, tb)
                start_out(j, tb)
                bn = (b + _LOOKAHEAD) % _NBUF
                if b < _LOOKAHEAD:
                    start_gather(j + _LOOKAHEAD, bn)
                else:
                    @pl.when(g < n_groups - 1)
                    def _():
                        start_gather(j + _LOOKAHEAD, bn)
            return carry

        lax.fori_loop(0, n_groups, group, 0)

        jg = (n_groups - 1) * _NBUF
        for b in range(_NBUF - _NTB, _NBUF):
            wait_out(jg + b, b % _NTB)

    return gather


def kernel(x, word_embed):
    batch, hist = x.shape
    vocab, embed = word_embed.shape
    idx = x.astype(jnp.int32).reshape(_NW, _CHUNK, hist).transpose(0, 2, 1)
    out5 = _make_gather(batch, hist, embed)(idx, word_embed)
    return out5.transpose(2, 4, 0, 1, 3).reshape(batch, hist, embed)
